# Initial kernel scaffold; baseline (speedup 1.0000x reference)
import jax
import jax.numpy as jnp
from jax.experimental import pallas as pl


def kernel(images, W_feat, b_feat, W_obj, b_obj, W_box, b_box):
    raise NotImplementedError("write your pallas kernel here")


# speedup vs baseline: 72.4665x; 72.4665x over previous
import jax
import jax.numpy as jnp
import numpy as np
from jax.experimental import pallas as pl
from jax.experimental.pallas import tpu as pltpu

_B = 2
_H = 640
_W = 640
_STRIDE = 16
_FH = _H // _STRIDE
_FW = _W // _STRIDE
_A = 3
_N = _FH * _FW * _A
_CF = 64
_IOU_THRESH = 0.5
_SCORE_THRESH = 0.001
_MAX_DET = 1000


def _make_anchors():
    size = 32.0
    ratios = np.array([0.5, 1.0, 2.0], dtype=np.float64)
    ws = size / np.sqrt(ratios)
    hs = size * np.sqrt(ratios)
    cy = (np.arange(_FH) + 0.5) * _STRIDE
    cx = (np.arange(_FW) + 0.5) * _STRIDE
    cyg, cxg = np.meshgrid(cy, cx, indexing='ij')
    cxg = cxg.reshape(-1, 1)
    cyg = cyg.reshape(-1, 1)
    x1 = cxg - ws[None, :] / 2.0
    y1 = cyg - hs[None, :] / 2.0
    x2 = cxg + ws[None, :] / 2.0
    y2 = cyg + hs[None, :] / 2.0
    anc = np.stack([x1, y1, x2, y2], axis=-1).reshape(-1, 4)
    return anc.astype(np.float32)


_ANCHORS = _make_anchors()

_T = 512
_NBLK = 10
_NPAD = _NBLK * _T


def _nms_body(sb_ref, tb_ref, keep_ref):
    bi = pl.program_id(0)

    @pl.when(bi == 0)
    def _init():
        keep_ref[...] = jnp.ones((_NBLK, _T), jnp.float32)

    cur = sb_ref[...]
    x1c, y1c = cur[:, 0:1], cur[:, 1:2]
    x2c, y2c = cur[:, 2:3], cur[:, 3:4]
    areac = (x2c - x1c) * (y2c - y1c)
    allb = tb_ref[...]

    def sup_mat(cj_start, cols4):
        x1a, y1a = cols4[0:1, :], cols4[1:2, :]
        x2a, y2a = cols4[2:3, :], cols4[3:4, :]
        areaa = (x2a - x1a) * (y2a - y1a)
        xx1 = jnp.maximum(x1c, x1a)
        yy1 = jnp.maximum(y1c, y1a)
        xx2 = jnp.minimum(x2c, x2a)
        yy2 = jnp.minimum(y2c, y2a)
        inter = jnp.maximum(xx2 - xx1, 0.0) * jnp.maximum(yy2 - yy1, 0.0)
        union = areac + areaa - inter + 1e-8
        over = (2.0 * inter - union) > union * float(2.0 ** -24)
        rows = bi * _T + jax.lax.broadcasted_iota(jnp.int32, (_T, _T), 0)
        cols = cj_start + jax.lax.broadcasted_iota(jnp.int32, (_T, _T), 1)
        return jnp.where(over & (cols > rows), 1.0, 0.0)

    cur_cols = tb_ref[:, pl.ds(bi * _T, _T)]
    m_blk = sup_mat(bi * _T, cur_cols)
    cand = keep_ref[pl.ds(bi, 1), :]

    def _cond(st):
        return st[1]

    def _body(st):
        k, _ = st
        supp = jnp.dot(k, m_blk, preferred_element_type=jnp.float32)
        kn = cand * jnp.where(supp < 0.5, 1.0, 0.0)
        return kn, jnp.any(kn != k)

    kfin, _ = jax.lax.while_loop(_cond, _body, (cand, jnp.bool_(True)))

    for cj in range(_NBLK):
        @pl.when(bi <= cj)
        def _chunk(cj=cj):
            cols4 = allb[:, cj * _T:(cj + 1) * _T]
            m_cj = sup_mat(cj * _T, cols4)
            supp = jnp.dot(kfin, m_cj, preferred_element_type=jnp.float32)
            old = keep_ref[cj:cj + 1, :]
            keep_ref[cj:cj + 1, :] = old * jnp.where(supp < 0.5, 1.0, 0.0)


def _nms_keep_single(b):
    bp = jnp.concatenate([b, jnp.zeros((_NPAD - _N, 4), jnp.float32)], axis=0)
    bt = jnp.transpose(bp, (1, 0))
    keep = pl.pallas_call(
        _nms_body,
        grid=(_NBLK,),
        in_specs=[
            pl.BlockSpec((_T, 4), lambda i: (i, 0)),
            pl.BlockSpec((4, _NPAD), lambda i: (0, 0)),
        ],
        out_specs=pl.BlockSpec((_NBLK, _T), lambda i: (0, 0)),
        out_shape=jax.ShapeDtypeStruct((_NBLK, _T), jnp.float32),
    )(bp, bt)
    return keep.reshape(_NPAD)[:_N] > 0.5


def _nms_single(boxes, scores):
    order = jnp.argsort(-scores)
    b = jnp.take(boxes, order, axis=0)
    s = jnp.take(scores, order)
    keep = _nms_keep_single(jax.lax.stop_gradient(b))
    s_kept = jnp.where(keep & (s > _SCORE_THRESH), s, -1.0)
    top_s, top_i = jax.lax.top_k(s_kept, _MAX_DET)
    top_b = jnp.take(b, top_i, axis=0)
    return jnp.concatenate([top_b, top_s[:, None]], axis=-1)


def kernel(images, W_feat, b_feat, W_obj, b_obj, W_box, b_box):
    dn = ('NCHW', 'OIHW', 'NCHW')
    feat = jax.lax.conv_general_dilated(
        images, W_feat, (_STRIDE, _STRIDE), 'VALID', dimension_numbers=dn)
    feat = jax.nn.relu(feat + b_feat[None, :, None, None])
    obj = jax.lax.conv_general_dilated(
        feat, W_obj, (1, 1), 'VALID', dimension_numbers=dn) + b_obj[None, :, None, None]
    box = jax.lax.conv_general_dilated(
        feat, W_box, (1, 1), 'VALID', dimension_numbers=dn) + b_box[None, :, None, None]
    obj = jnp.transpose(obj, (0, 2, 3, 1)).reshape(_B, _N)
    box = jnp.transpose(box, (0, 2, 3, 1)).reshape(_B, _N, 4)
    anc = jnp.asarray(_ANCHORS)
    aw = anc[:, 2] - anc[:, 0]
    ah = anc[:, 3] - anc[:, 1]
    acx = anc[:, 0] + 0.5 * aw
    acy = anc[:, 1] + 0.5 * ah
    dx = box[..., 0]
    dy = box[..., 1]
    dw = jnp.clip(box[..., 2], -4.0, 4.0)
    dh = jnp.clip(box[..., 3], -4.0, 4.0)
    pcx = acx[None, :] + dx * aw[None, :]
    pcy = acy[None, :] + dy * ah[None, :]
    pw = aw[None, :] * jnp.exp(dw)
    ph = ah[None, :] * jnp.exp(dh)
    x1 = jnp.clip(pcx - 0.5 * pw, 0.0, float(_W))
    y1 = jnp.clip(pcy - 0.5 * ph, 0.0, float(_H))
    x2 = jnp.clip(pcx + 0.5 * pw, 0.0, float(_W))
    y2 = jnp.clip(pcy + 0.5 * ph, 0.0, float(_H))
    boxes = jnp.stack([x1, y1, x2, y2], axis=-1)
    scores = jax.nn.sigmoid(obj)
    return jax.vmap(_nms_single)(boxes, scores)
